# baseline (speedup 1.0000x reference)
import jax
import jax.numpy as jnp
from jax import lax
from jax.experimental import pallas as pl
from jax.experimental.pallas import tpu as pltpu
from jax.experimental.pallas import tpu_sc as plsc

BATCH = 16384
DIM = 64
LANES = 16
NUM_WORKERS = 32
BPW = BATCH // NUM_WORKERS
G = 128
NST = BPW // G


def _body(s_hbm, r_hbm, d_hbm, emb_hbm, rel_hbm, out_hbm,
          sidx, ridx, didx, hbuf, rbuf, tbuf, outv, sem):
    cid = lax.axis_index("c")
    sid = lax.axis_index("s")
    wid = sid * 2 + cid
    base = wid * BPW

    lanes = lax.iota(jnp.int32, LANES)

    pltpu.sync_copy(s_hbm.at[pl.ds(base, BPW)], sidx)
    pltpu.sync_copy(r_hbm.at[pl.ds(base, BPW)], ridx)
    pltpu.sync_copy(d_hbm.at[pl.ds(base, BPW)], didx)

    def fire(stage, b):
        def fire_grp(g, carry):
            off = stage * G + g * LANES
            vs = sidx[pl.ds(off, LANES)]
            vr = ridx[pl.ds(off, LANES)]
            vd = didx[pl.ds(off, LANES)]
            for k in range(LANES):
                row = g * LANES + k
                pltpu.async_copy(emb_hbm.at[vs[k]], hbuf.at[b, row], sem)
                pltpu.async_copy(rel_hbm.at[vr[k]], rbuf.at[b, row], sem)
                pltpu.async_copy(emb_hbm.at[vd[k]], tbuf.at[b, row], sem)
            return carry

        lax.fori_loop(0, G // LANES, fire_grp, 0)

    def drain(b):
        pltpu.make_async_copy(emb_hbm.at[pl.ds(0, G)], hbuf.at[b], sem).wait()
        pltpu.make_async_copy(emb_hbm.at[pl.ds(0, G)], rbuf.at[b], sem).wait()
        pltpu.make_async_copy(emb_hbm.at[pl.ds(0, G)], tbuf.at[b], sem).wait()

    def compute(stage, b):
        def cg(g, carry):
            acc = jnp.zeros((LANES,), jnp.float32)
            for k in range(LANES):
                row = g * LANES + k
                w = jnp.zeros((LANES,), jnp.float32)
                for j in range(DIM // LANES):
                    sl = pl.ds(j * LANES, LANES)
                    h = hbuf[b, row, sl]
                    rr = rbuf[b, row, sl]
                    t = tbuf[b, row, sl]
                    w = w + jnp.abs(h + rr - t)
                acc = jnp.where(lanes == k, jnp.sum(w), acc)
            outv[pl.ds(stage * G + g * LANES, LANES)] = 1.0 / (1.0 + jnp.exp(acc))
            return carry

        lax.fori_loop(0, G // LANES, cg, 0)

    fire(0, 0)

    def stage_loop(s, carry):
        b = lax.rem(s, 2)
        drain(b)

        @pl.when(s < NST - 1)
        def _():
            fire(s + 1, 1 - b)

        compute(s, b)
        return carry

    lax.fori_loop(0, NST, stage_loop, 0)

    pltpu.sync_copy(outv, out_hbm.at[pl.ds(base, BPW)])


def kernel(x, emb, rel_emb):
    mesh = plsc.VectorSubcoreMesh(core_axis_name="c", subcore_axis_name="s")
    run = pl.kernel(
        _body,
        out_type=jax.ShapeDtypeStruct((BATCH,), jnp.float32),
        mesh=mesh,
        compiler_params=pltpu.CompilerParams(needs_layout_passes=False),
        scratch_types=[
            pltpu.VMEM((BPW,), jnp.int32),
            pltpu.VMEM((BPW,), jnp.int32),
            pltpu.VMEM((BPW,), jnp.int32),
            pltpu.VMEM((2, G, DIM), jnp.float32),
            pltpu.VMEM((2, G, DIM), jnp.float32),
            pltpu.VMEM((2, G, DIM), jnp.float32),
            pltpu.VMEM((BPW,), jnp.float32),
            pltpu.SemaphoreType.DMA,
        ],
    )
    xi = x.astype(jnp.int32)
    return run(xi[:, 0], xi[:, 1], xi[:, 2], emb, rel_emb)


# speedup vs baseline: 1.0007x; 1.0007x over previous
import jax
import jax.numpy as jnp
from jax import lax
from jax.experimental import pallas as pl
from jax.experimental.pallas import tpu as pltpu
from jax.experimental.pallas import tpu_sc as plsc

BATCH = 16384
DIM = 64
LANES = 16
NUM_WORKERS = 32
BPW = BATCH // NUM_WORKERS
G = 128
NST = BPW // G


def _body(s_hbm, r_hbm, d_hbm, emb_hbm, rel_hbm, out_hbm,
          sidx, ridx, didx, hbuf, rbuf, tbuf, outv, sem, sem2, sem3):
    cid = lax.axis_index("c")
    sid = lax.axis_index("s")
    wid = sid * 2 + cid
    base = wid * BPW

    lanes = lax.iota(jnp.int32, LANES)

    pltpu.sync_copy(s_hbm.at[pl.ds(base, BPW)], sidx)
    pltpu.sync_copy(r_hbm.at[pl.ds(base, BPW)], ridx)
    pltpu.sync_copy(d_hbm.at[pl.ds(base, BPW)], didx)

    def fire(stage, b):
        def fire_grp(g, carry):
            off = stage * G + g * LANES
            vs = sidx[pl.ds(off, LANES)]
            vr = ridx[pl.ds(off, LANES)]
            vd = didx[pl.ds(off, LANES)]
            for k in range(LANES):
                row = g * LANES + k
                pltpu.async_copy(emb_hbm.at[vs[k]], hbuf.at[b, row], sem)
                pltpu.async_copy(rel_hbm.at[vr[k]], rbuf.at[b, row], sem2)
                pltpu.async_copy(emb_hbm.at[vd[k]], tbuf.at[b, row], sem3)
            return carry

        lax.fori_loop(0, G // LANES, fire_grp, 0)

    def drain(b):
        pltpu.make_async_copy(emb_hbm.at[pl.ds(0, G)], hbuf.at[b], sem).wait()
        pltpu.make_async_copy(emb_hbm.at[pl.ds(0, G)], rbuf.at[b], sem2).wait()
        pltpu.make_async_copy(emb_hbm.at[pl.ds(0, G)], tbuf.at[b], sem3).wait()

    def compute(stage, b):
        def cg(g, carry):
            acc = jnp.zeros((LANES,), jnp.float32)
            for k in range(LANES):
                row = g * LANES + k
                w = jnp.zeros((LANES,), jnp.float32)
                for j in range(DIM // LANES):
                    sl = pl.ds(j * LANES, LANES)
                    h = hbuf[b, row, sl]
                    rr = rbuf[b, row, sl]
                    t = tbuf[b, row, sl]
                    w = w + jnp.abs(h + rr - t)
                acc = jnp.where(lanes == k, jnp.sum(w), acc)
            outv[pl.ds(stage * G + g * LANES, LANES)] = 1.0 / (1.0 + jnp.exp(acc))
            return carry

        lax.fori_loop(0, G // LANES, cg, 0)

    fire(0, 0)

    def stage_loop(s, carry):
        b = lax.rem(s, 2)
        drain(b)

        @pl.when(s < NST - 1)
        def _():
            fire(s + 1, 1 - b)

        compute(s, b)
        return carry

    lax.fori_loop(0, NST, stage_loop, 0)

    pltpu.sync_copy(outv, out_hbm.at[pl.ds(base, BPW)])


def kernel(x, emb, rel_emb):
    mesh = plsc.VectorSubcoreMesh(core_axis_name="c", subcore_axis_name="s")
    run = pl.kernel(
        _body,
        out_type=jax.ShapeDtypeStruct((BATCH,), jnp.float32),
        mesh=mesh,
        compiler_params=pltpu.CompilerParams(needs_layout_passes=False),
        scratch_types=[
            pltpu.VMEM((BPW,), jnp.int32),
            pltpu.VMEM((BPW,), jnp.int32),
            pltpu.VMEM((BPW,), jnp.int32),
            pltpu.VMEM((2, G, DIM), jnp.float32),
            pltpu.VMEM((2, G, DIM), jnp.float32),
            pltpu.VMEM((2, G, DIM), jnp.float32),
            pltpu.VMEM((BPW,), jnp.float32),
            pltpu.SemaphoreType.DMA,
            pltpu.SemaphoreType.DMA,
            pltpu.SemaphoreType.DMA,
        ],
    )
    xi = x.astype(jnp.int32)
    return run(xi[:, 0], xi[:, 1], xi[:, 2], emb, rel_emb)
